# baseline (speedup 1.0000x reference)
import jax
import jax.numpy as jnp
from jax import lax
from jax.experimental import pallas as pl
from jax.experimental.pallas import tpu as pltpu
from jax.experimental.pallas import tpu_sc as plsc

N = 10000
DIN = 128
HEADS = 8
DH = 8
F1 = HEADS * DH
DOUT = 16

NC, NS = 2, 16
NW = NC * NS
SLAB = 632
NPAD = NS * SLAB
CHUNK = 128
K = 82
TPW = CHUNK * K
EPAD = NW * TPW
TOTCH = EPAD // CHUNK
W1R = 80
W2R = 32


def _tc1_body(x_ref, g1_ref, gad_ref, s_ref, ad_ref):
    xb = x_ref[...]
    s_ref[...] = jnp.dot(xb, g1_ref[...], preferred_element_type=jnp.float32)
    ad_ref[...] = jnp.dot(xb, gad_ref[...], preferred_element_type=jnp.float32)


def _tc2_body(pa_ref, pb_ref, r_ref, b1_ref, g2_ref, gad2_ref, s2_ref, ad2_ref):
    A = pa_ref[...] + pb_ref[...]
    dexp = jnp.dot(A[:, 0:8], r_ref[...], preferred_element_type=jnp.float32)
    out1 = jnp.maximum(A[:, 16:80] / (dexp + 1e-16) + b1_ref[...], 0.0)
    s2_ref[...] = jnp.dot(out1, g2_ref[...], preferred_element_type=jnp.float32)
    ad2_ref[...] = jnp.dot(out1, gad2_ref[...], preferred_element_type=jnp.float32)


def _tc3_body(pa_ref, pb_ref, b2_ref, o_ref):
    A = pa_ref[...] + pb_ref[...]
    o2 = A[:, 16:32] / (A[:, 0:1] + 1e-16) + b2_ref[...]
    z = o2 - jnp.max(o2, axis=1, keepdims=True)
    o_ref[...] = z - jnp.log(jnp.sum(jnp.exp(z), axis=1, keepdims=True))


def _make_sc_body(width, compute_rows):
    nslice = width // 16

    def body(src_hbm, dst_hbm, s_hbm, ad_hbm, out_hbm,
             idxs, idxd, rows0, ad0, rows1, ad1, ov, accum, adspm,
             sg0, sa0, sg1, sa1):
        cid = lax.axis_index("c")
        sid = lax.axis_index("s")
        wid = sid * NC + cid
        kbase = wid * K
        z16 = jnp.zeros((16,), jnp.float32)

        pltpu.sync_copy(src_hbm.at[pl.ds(kbase, K)], idxs)
        pltpu.sync_copy(dst_hbm.at[pl.ds(kbase, K)], idxd)
        pltpu.sync_copy(ad_hbm.at[pl.ds(sid * SLAB, SLAB)],
                        adspm.at[pl.ds(sid * SLAB, SLAB)])
        pltpu.async_copy(s_hbm.at[idxs.at[0]], rows0, sg0)

        @plsc.parallel_loop(0, CHUNK, unroll=8)
        def _zrow(r):
            for j in range(nslice):
                ov[r, pl.ds(j * 16, 16)] = z16
        for piece in range(5):
            rows_n = 128 if piece < 4 else 120
            pltpu.sync_copy(ov.at[pl.ds(0, rows_n)],
                            accum.at[pl.ds(sid * SLAB + piece * 128, rows_n)])
        plsc.subcore_barrier()
        pltpu.async_copy(adspm.at[idxd.at[0]], ad0, sa0)

        bufs = ((rows0, ad0, sg0, sa0), (rows1, ad1, sg1, sa1))

        def _pair(k2, carry):
            k0 = k2 * 2
            for b in range(2):
                kk = k0 + b
                rb, ab, sg, sa = bufs[b]
                nrb, nab, nsg, nsa = bufs[1 - b]

                @pl.when(kk + 1 < K)
                def _issue():
                    pltpu.async_copy(s_hbm.at[idxs.at[kk + 1]], nrb, nsg)
                    pltpu.async_copy(adspm.at[idxd.at[kk + 1]], nab, nsa)

                pltpu.make_async_copy(s_hbm.at[idxs.at[kk]], rb, sg).wait()
                pltpu.make_async_copy(adspm.at[idxd.at[kk]], ab, sa).wait()

                @plsc.parallel_loop(0, CHUNK, unroll=8)
                def _edge(i):
                    compute_rows(rb, ab, ov, i)
                pltpu.sync_copy(ov, accum.at[idxd.at[kk]], add=True)
            return carry

        lax.fori_loop(0, K // 2, _pair, 0)
        plsc.subcore_barrier()
        for piece in range(5):
            rows_n = 128 if piece < 4 else 120
            pltpu.sync_copy(
                accum.at[pl.ds(sid * SLAB + piece * 128, rows_n)],
                out_hbm.at[pl.ds(cid * NPAD + sid * SLAB + piece * 128, rows_n)])

    return body


def _compute_rows_l1(rows, adv, ov, i):
    aa = rows[i, pl.ds(0, 16)] + adv[i, pl.ds(0, 16)]
    p = jnp.exp(jnp.maximum(aa, 0.2 * aa))
    ov[i, pl.ds(0, 16)] = p
    for j in range(4):
        ov[i, pl.ds(16 + 16 * j, 16)] = rows[i, pl.ds(16 + 16 * j, 16)] * p


def _compute_rows_l2(rows, adv, ov, i):
    aa = rows[i, pl.ds(0, 16)] + adv[i, pl.ds(0, 16)]
    p = jnp.exp(jnp.maximum(aa, 0.2 * aa))
    ov[i, pl.ds(0, 16)] = p
    ov[i, pl.ds(16, 16)] = rows[i, pl.ds(16, 16)] * p


def _make_sc(width, compute_rows):
    return pl.kernel(
        _make_sc_body(width, compute_rows),
        out_type=jax.ShapeDtypeStruct((NC * NPAD, width), jnp.float32),
        mesh=plsc.VectorSubcoreMesh(core_axis_name="c", subcore_axis_name="s",
                                    num_cores=NC, num_subcores=NS),
        scratch_types=[
            pltpu.VMEM((K, CHUNK), jnp.int32),
            pltpu.VMEM((K, CHUNK), jnp.int32),
            pltpu.VMEM((CHUNK, width), jnp.float32),
            pltpu.VMEM((CHUNK, 16), jnp.float32),
            pltpu.VMEM((CHUNK, width), jnp.float32),
            pltpu.VMEM((CHUNK, 16), jnp.float32),
            pltpu.VMEM((CHUNK, width), jnp.float32),
            pltpu.VMEM_SHARED((NPAD, width), jnp.float32),
            pltpu.VMEM_SHARED((NPAD, 16), jnp.float32),
            pltpu.SemaphoreType.DMA,
            pltpu.SemaphoreType.DMA,
            pltpu.SemaphoreType.DMA,
            pltpu.SemaphoreType.DMA,
        ],
        compiler_params=pltpu.CompilerParams(use_tc_tiling_on_sc=False),
    )


_GRID = NPAD // SLAB


def kernel(x, edge_index, W1, att_src1, att_dst1, b1, W2, att_src2, att_dst2, b2):
    f32 = jnp.float32
    lanes = jnp.arange(F1)
    A_src = jnp.zeros((F1, HEADS), f32).at[lanes, lanes // DH].set(att_src1.reshape(-1))
    A_dst = jnp.zeros((F1, HEADS), f32).at[lanes, lanes // DH].set(att_dst1.reshape(-1))
    t = jnp.arange(F1)
    perm = (t % DH) * HEADS + t // DH
    P1m = jnp.zeros((F1, F1), f32).at[perm, t].set(1.0)
    M1 = jnp.concatenate([A_src, A_src, P1m], axis=1)
    G1 = W1 @ M1
    GAD1 = W1 @ jnp.concatenate([A_dst, A_dst], axis=1)
    ones16 = jnp.ones((1, 16), f32)
    M2 = jnp.concatenate([att_src2.reshape(DOUT, 1) @ ones16,
                          jnp.eye(DOUT, dtype=f32)], axis=1)
    G2 = W2[perm, :] @ M2
    GAD2 = W2[perm, :] @ (att_dst2.reshape(DOUT, 1) @ ones16)
    R = jnp.zeros((HEADS, F1), f32).at[t % DH, t].set(1.0)
    b1r = b1[perm].reshape(1, F1)
    b2r = b2.reshape(1, DOUT)

    loop = jnp.arange(N, dtype=jnp.int32)
    fill = jnp.full((EPAD - N - edge_index.shape[1],), N, jnp.int32)
    src = jnp.concatenate([edge_index[0], loop, fill]).reshape(TOTCH, CHUNK)
    dst = jnp.concatenate([edge_index[1], loop, fill]).reshape(TOTCH, CHUNK)
    xp = jnp.pad(x, ((0, NPAD - N), (0, 0)))

    S1, AD1 = pl.pallas_call(
        _tc1_body,
        grid=(_GRID,),
        in_specs=[pl.BlockSpec((SLAB, DIN), lambda i: (i, 0)),
                  pl.BlockSpec((DIN, W1R), lambda i: (0, 0)),
                  pl.BlockSpec((DIN, 16), lambda i: (0, 0))],
        out_specs=[pl.BlockSpec((SLAB, W1R), lambda i: (i, 0)),
                   pl.BlockSpec((SLAB, 16), lambda i: (i, 0))],
        out_shape=[jax.ShapeDtypeStruct((NPAD, W1R), f32),
                   jax.ShapeDtypeStruct((NPAD, 16), f32)],
    )(xp, G1, GAD1)

    P1 = _make_sc(W1R, _compute_rows_l1)(src, dst, S1, AD1)

    S2, AD2 = pl.pallas_call(
        _tc2_body,
        grid=(_GRID,),
        in_specs=[pl.BlockSpec((SLAB, W1R), lambda i: (i, 0)),
                  pl.BlockSpec((SLAB, W1R), lambda i: (i + _GRID, 0)),
                  pl.BlockSpec((HEADS, F1), lambda i: (0, 0)),
                  pl.BlockSpec((1, F1), lambda i: (0, 0)),
                  pl.BlockSpec((F1, W2R), lambda i: (0, 0)),
                  pl.BlockSpec((F1, 16), lambda i: (0, 0))],
        out_specs=[pl.BlockSpec((SLAB, W2R), lambda i: (i, 0)),
                   pl.BlockSpec((SLAB, 16), lambda i: (i, 0))],
        out_shape=[jax.ShapeDtypeStruct((NPAD, W2R), f32),
                   jax.ShapeDtypeStruct((NPAD, 16), f32)],
    )(P1, P1, R, b1r, G2, GAD2)

    P2 = _make_sc(W2R, _compute_rows_l2)(src, dst, S2, AD2)

    out = pl.pallas_call(
        _tc3_body,
        grid=(_GRID,),
        in_specs=[pl.BlockSpec((SLAB, W2R), lambda i: (i, 0)),
                  pl.BlockSpec((SLAB, W2R), lambda i: (i + _GRID, 0)),
                  pl.BlockSpec((1, DOUT), lambda i: (0, 0))],
        out_specs=pl.BlockSpec((SLAB, DOUT), lambda i: (i, 0)),
        out_shape=jax.ShapeDtypeStruct((NPAD, DOUT), f32),
    )(P2, P2, b2r)
    return out[:N]


# speedup vs baseline: 1.0528x; 1.0528x over previous
import jax
import jax.numpy as jnp
from jax import lax
from jax.experimental import pallas as pl
from jax.experimental.pallas import tpu as pltpu
from jax.experimental.pallas import tpu_sc as plsc

N = 10000
DIN = 128
HEADS = 8
DH = 8
F1 = HEADS * DH
DOUT = 16

NC, NS = 2, 16
NW = NC * NS
SLAB = 632
NPAD = NS * SLAB
CHUNK = 64
NBUF = 4
K = 164
TPW = CHUNK * K
EPAD = NW * TPW
TOTCH = EPAD // CHUNK
W1R = 80
W2R = 32


def _tc1_body(x_ref, g1_ref, gad_ref, s_ref, ad_ref):
    xb = x_ref[...]
    s_ref[...] = jnp.dot(xb, g1_ref[...], preferred_element_type=jnp.float32)
    ad_ref[...] = jnp.dot(xb, gad_ref[...], preferred_element_type=jnp.float32)


def _tc2_body(pa_ref, pb_ref, r_ref, b1_ref, g2_ref, gad2_ref, s2_ref, ad2_ref):
    A = pa_ref[...] + pb_ref[...]
    dexp = jnp.dot(A[:, 0:8], r_ref[...], preferred_element_type=jnp.float32)
    out1 = jnp.maximum(A[:, 16:80] / (dexp + 1e-16) + b1_ref[...], 0.0)
    s2_ref[...] = jnp.dot(out1, g2_ref[...], preferred_element_type=jnp.float32)
    ad2_ref[...] = jnp.dot(out1, gad2_ref[...], preferred_element_type=jnp.float32)


def _tc3_body(pa_ref, pb_ref, b2_ref, o_ref):
    A = pa_ref[...] + pb_ref[...]
    o2 = A[:, 16:32] / (A[:, 0:1] + 1e-16) + b2_ref[...]
    z = o2 - jnp.max(o2, axis=1, keepdims=True)
    o_ref[...] = z - jnp.log(jnp.sum(jnp.exp(z), axis=1, keepdims=True))


def _make_sc_body(width, compute_rows):
    nslice = width // 16

    def body(src_hbm, dst_hbm, s_hbm, ad_hbm, out_hbm,
             idxs, idxd, rows0, ad0, rows1, ad1, rows2, ad2, rows3, ad3,
             ov, accum, sg0, sa0, sg1, sa1, sg2, sa2, sg3, sa3):
        cid = lax.axis_index("c")
        sid = lax.axis_index("s")
        wid = sid * NC + cid
        kbase = wid * K
        z16 = jnp.zeros((16,), jnp.float32)

        pltpu.sync_copy(src_hbm.at[pl.ds(kbase, K)], idxs)
        pltpu.sync_copy(dst_hbm.at[pl.ds(kbase, K)], idxd)
        pltpu.async_copy(s_hbm.at[idxs.at[0]], rows0, sg0)
        pltpu.async_copy(ad_hbm.at[idxd.at[0]], ad0, sa0)

        @plsc.parallel_loop(0, CHUNK, unroll=8)
        def _zrow(r):
            for j in range(nslice):
                ov[r, pl.ds(j * 16, 16)] = z16
        for piece in range(10):
            rows_n = 64 if piece < 9 else 56
            pltpu.sync_copy(ov.at[pl.ds(0, rows_n)],
                            accum.at[pl.ds(sid * SLAB + piece * 64, rows_n)])
        plsc.subcore_barrier()

        bufs = ((rows0, ad0, sg0, sa0), (rows1, ad1, sg1, sa1),
                (rows2, ad2, sg2, sa2), (rows3, ad3, sg3, sa3))
        for b in range(1, NBUF - 1):
            rb, ab, sg, sa = bufs[b]
            pltpu.async_copy(s_hbm.at[idxs.at[b]], rb, sg)
            pltpu.async_copy(ad_hbm.at[idxd.at[b]], ab, sa)

        def _ring(kq, carry):
            k0 = kq * NBUF
            for b in range(NBUF):
                kk = k0 + b
                rb, ab, sg, sa = bufs[b]
                nrb, nab, nsg, nsa = bufs[(b + NBUF - 1) % NBUF]

                @pl.when(kk + NBUF - 1 < K)
                def _issue():
                    pltpu.async_copy(s_hbm.at[idxs.at[kk + NBUF - 1]], nrb, nsg)
                    pltpu.async_copy(ad_hbm.at[idxd.at[kk + NBUF - 1]], nab, nsa)

                pltpu.make_async_copy(s_hbm.at[idxs.at[kk]], rb, sg).wait()
                pltpu.make_async_copy(ad_hbm.at[idxd.at[kk]], ab, sa).wait()

                @plsc.parallel_loop(0, CHUNK, unroll=8)
                def _edge(i):
                    compute_rows(rb, ab, ov, i)

                pltpu.sync_copy(ov, accum.at[idxd.at[kk]], add=True)
            return carry

        lax.fori_loop(0, K // NBUF, _ring, 0)
        plsc.subcore_barrier()
        for piece in range(5):
            rows_n = 128 if piece < 4 else 120
            pltpu.sync_copy(
                accum.at[pl.ds(sid * SLAB + piece * 128, rows_n)],
                out_hbm.at[pl.ds(cid * NPAD + sid * SLAB + piece * 128, rows_n)])

    return body


def _compute_rows_l1(rows, adv, ov, i):
    aa = rows[i, pl.ds(0, 16)] + adv[i, pl.ds(0, 16)]
    p = jnp.exp(jnp.maximum(aa, 0.2 * aa))
    ov[i, pl.ds(0, 16)] = p
    for j in range(4):
        ov[i, pl.ds(16 + 16 * j, 16)] = rows[i, pl.ds(16 + 16 * j, 16)] * p


def _compute_rows_l2(rows, adv, ov, i):
    aa = rows[i, pl.ds(0, 16)] + adv[i, pl.ds(0, 16)]
    p = jnp.exp(jnp.maximum(aa, 0.2 * aa))
    ov[i, pl.ds(0, 16)] = p
    ov[i, pl.ds(16, 16)] = rows[i, pl.ds(16, 16)] * p


def _make_sc(width, compute_rows):
    return pl.kernel(
        _make_sc_body(width, compute_rows),
        out_type=jax.ShapeDtypeStruct((NC * NPAD, width), jnp.float32),
        mesh=plsc.VectorSubcoreMesh(core_axis_name="c", subcore_axis_name="s",
                                    num_cores=NC, num_subcores=NS),
        scratch_types=[
            pltpu.VMEM((K, CHUNK), jnp.int32),
            pltpu.VMEM((K, CHUNK), jnp.int32),
            pltpu.VMEM((CHUNK, width), jnp.float32),
            pltpu.VMEM((CHUNK, 16), jnp.float32),
            pltpu.VMEM((CHUNK, width), jnp.float32),
            pltpu.VMEM((CHUNK, 16), jnp.float32),
            pltpu.VMEM((CHUNK, width), jnp.float32),
            pltpu.VMEM((CHUNK, 16), jnp.float32),
            pltpu.VMEM((CHUNK, width), jnp.float32),
            pltpu.VMEM((CHUNK, 16), jnp.float32),
            pltpu.VMEM((CHUNK, width), jnp.float32),
            pltpu.VMEM_SHARED((NPAD, width), jnp.float32),
            pltpu.SemaphoreType.DMA,
            pltpu.SemaphoreType.DMA,
            pltpu.SemaphoreType.DMA,
            pltpu.SemaphoreType.DMA,
            pltpu.SemaphoreType.DMA,
            pltpu.SemaphoreType.DMA,
            pltpu.SemaphoreType.DMA,
            pltpu.SemaphoreType.DMA,
        ],
        compiler_params=pltpu.CompilerParams(use_tc_tiling_on_sc=False),
    )


_GRID = NPAD // SLAB


def kernel(x, edge_index, W1, att_src1, att_dst1, b1, W2, att_src2, att_dst2, b2):
    f32 = jnp.float32
    lanes = jnp.arange(F1)
    A_src = jnp.zeros((F1, HEADS), f32).at[lanes, lanes // DH].set(att_src1.reshape(-1))
    A_dst = jnp.zeros((F1, HEADS), f32).at[lanes, lanes // DH].set(att_dst1.reshape(-1))
    t = jnp.arange(F1)
    perm = (t % DH) * HEADS + t // DH
    P1m = jnp.zeros((F1, F1), f32).at[perm, t].set(1.0)
    M1 = jnp.concatenate([A_src, A_src, P1m], axis=1)
    G1 = W1 @ M1
    GAD1 = W1 @ jnp.concatenate([A_dst, A_dst], axis=1)
    ones16 = jnp.ones((1, 16), f32)
    M2 = jnp.concatenate([att_src2.reshape(DOUT, 1) @ ones16,
                          jnp.eye(DOUT, dtype=f32)], axis=1)
    G2 = W2[perm, :] @ M2
    GAD2 = W2[perm, :] @ (att_dst2.reshape(DOUT, 1) @ ones16)
    R = jnp.zeros((HEADS, F1), f32).at[t % DH, t].set(1.0)
    b1r = b1[perm].reshape(1, F1)
    b2r = b2.reshape(1, DOUT)

    loop = jnp.arange(N, dtype=jnp.int32)
    fill = jnp.full((EPAD - N - edge_index.shape[1],), N, jnp.int32)
    src = jnp.concatenate([edge_index[0], loop, fill]).reshape(TOTCH, CHUNK)
    dst = jnp.concatenate([edge_index[1], loop, fill]).reshape(TOTCH, CHUNK)
    xp = jnp.pad(x, ((0, NPAD - N), (0, 0)))

    S1, AD1 = pl.pallas_call(
        _tc1_body,
        grid=(_GRID,),
        in_specs=[pl.BlockSpec((SLAB, DIN), lambda i: (i, 0)),
                  pl.BlockSpec((DIN, W1R), lambda i: (0, 0)),
                  pl.BlockSpec((DIN, 16), lambda i: (0, 0))],
        out_specs=[pl.BlockSpec((SLAB, W1R), lambda i: (i, 0)),
                   pl.BlockSpec((SLAB, 16), lambda i: (i, 0))],
        out_shape=[jax.ShapeDtypeStruct((NPAD, W1R), f32),
                   jax.ShapeDtypeStruct((NPAD, 16), f32)],
    )(xp, G1, GAD1)

    P1 = _make_sc(W1R, _compute_rows_l1)(src, dst, S1, AD1)

    S2, AD2 = pl.pallas_call(
        _tc2_body,
        grid=(_GRID,),
        in_specs=[pl.BlockSpec((SLAB, W1R), lambda i: (i, 0)),
                  pl.BlockSpec((SLAB, W1R), lambda i: (i + _GRID, 0)),
                  pl.BlockSpec((HEADS, F1), lambda i: (0, 0)),
                  pl.BlockSpec((1, F1), lambda i: (0, 0)),
                  pl.BlockSpec((F1, W2R), lambda i: (0, 0)),
                  pl.BlockSpec((F1, 16), lambda i: (0, 0))],
        out_specs=[pl.BlockSpec((SLAB, W2R), lambda i: (i, 0)),
                   pl.BlockSpec((SLAB, 16), lambda i: (i, 0))],
        out_shape=[jax.ShapeDtypeStruct((NPAD, W2R), f32),
                   jax.ShapeDtypeStruct((NPAD, 16), f32)],
    )(P1, P1, R, b1r, G2, GAD2)

    P2 = _make_sc(W2R, _compute_rows_l2)(src, dst, S2, AD2)

    out = pl.pallas_call(
        _tc3_body,
        grid=(_GRID,),
        in_specs=[pl.BlockSpec((SLAB, W2R), lambda i: (i, 0)),
                  pl.BlockSpec((SLAB, W2R), lambda i: (i + _GRID, 0)),
                  pl.BlockSpec((1, DOUT), lambda i: (0, 0))],
        out_specs=pl.BlockSpec((SLAB, DOUT), lambda i: (i, 0)),
        out_shape=jax.ShapeDtypeStruct((NPAD, DOUT), f32),
    )(P2, P2, b2r)
    return out[:N]


# speedup vs baseline: 1.3291x; 1.2624x over previous
import jax
import jax.numpy as jnp
from jax import lax
from jax.experimental import pallas as pl
from jax.experimental.pallas import tpu as pltpu
from jax.experimental.pallas import tpu_sc as plsc

N = 10000
DIN = 128
HEADS = 8
DH = 8
F1 = HEADS * DH
DOUT = 16

NC, NS = 2, 16
NW = NC * NS
SLAB = 632
NPAD = NS * SLAB
CHUNK = 64
NBUF = 4
K0 = 232
K1 = 96
PAIR = K0 + K1
TOTCH = NS * PAIR
EPAD = (NS * PAIR + K0) * CHUNK

W1R = 80
W2R = 32


def _tc1_body(x_ref, g1_ref, gad_ref, s_ref, ad_ref):
    xb = x_ref[...]
    s_ref[...] = jnp.dot(xb, g1_ref[...], preferred_element_type=jnp.float32)
    ad_ref[...] = jnp.dot(xb, gad_ref[...], preferred_element_type=jnp.float32)


def _tc2_body(pa_ref, pb_ref, r_ref, b1_ref, g2_ref, gad2_ref, s2_ref, ad2_ref):
    A = pa_ref[...] + pb_ref[...]
    dexp = jnp.dot(A[:, 0:8], r_ref[...], preferred_element_type=jnp.float32)
    out1 = jnp.maximum(A[:, 16:80] / (dexp + 1e-16) + b1_ref[...], 0.0)
    s2_ref[...] = jnp.dot(out1, g2_ref[...], preferred_element_type=jnp.float32)
    ad2_ref[...] = jnp.dot(out1, gad2_ref[...], preferred_element_type=jnp.float32)


def _tc3_body(pa_ref, pb_ref, b2_ref, o_ref):
    A = pa_ref[...] + pb_ref[...]
    o2 = A[:, 16:32] / (A[:, 0:1] + 1e-16) + b2_ref[...]
    z = o2 - jnp.max(o2, axis=1, keepdims=True)
    o_ref[...] = z - jnp.log(jnp.sum(jnp.exp(z), axis=1, keepdims=True))


def _make_sc_body(width, compute_rows):
    nslice = width // 16

    def body(src_hbm, dst_hbm, s_hbm, ad_hbm, out_hbm,
             idxs, idxd, rows0, ad0, rows1, ad1, rows2, ad2, rows3, ad3,
             ov, accum, sg0, sa0, sg1, sa1, sg2, sa2, sg3, sa3):
        cid = lax.axis_index("c")
        sid = lax.axis_index("s")
        kbase = sid * PAIR + cid * K0
        cnt = jnp.where(cid == 0, K0, K1)
        nq = jnp.where(cid == 0, K0 // NBUF, K1 // NBUF)
        z16 = jnp.zeros((16,), jnp.float32)

        pltpu.sync_copy(src_hbm.at[pl.ds(kbase, K0)], idxs)
        pltpu.sync_copy(dst_hbm.at[pl.ds(kbase, K0)], idxd)
        pltpu.async_copy(s_hbm.at[idxs.at[0]], rows0, sg0)
        pltpu.async_copy(ad_hbm.at[idxd.at[0]], ad0, sa0)

        @plsc.parallel_loop(0, CHUNK, unroll=8)
        def _zrow(r):
            for j in range(nslice):
                ov[r, pl.ds(j * 16, 16)] = z16
        for piece in range(10):
            rows_n = 64 if piece < 9 else 56
            pltpu.sync_copy(ov.at[pl.ds(0, rows_n)],
                            accum.at[pl.ds(sid * SLAB + piece * 64, rows_n)])
        plsc.subcore_barrier()

        bufs = ((rows0, ad0, sg0, sa0), (rows1, ad1, sg1, sa1),
                (rows2, ad2, sg2, sa2), (rows3, ad3, sg3, sa3))
        for b in range(1, NBUF - 1):
            rb, ab, sg, sa = bufs[b]
            pltpu.async_copy(s_hbm.at[idxs.at[b]], rb, sg)
            pltpu.async_copy(ad_hbm.at[idxd.at[b]], ab, sa)

        def _ring(kq, carry):
            k0 = kq * NBUF
            for b in range(NBUF):
                kk = k0 + b
                rb, ab, sg, sa = bufs[b]
                nrb, nab, nsg, nsa = bufs[(b + NBUF - 1) % NBUF]

                @pl.when(kk + NBUF - 1 < cnt)
                def _issue():
                    pltpu.async_copy(s_hbm.at[idxs.at[kk + NBUF - 1]], nrb, nsg)
                    pltpu.async_copy(ad_hbm.at[idxd.at[kk + NBUF - 1]], nab, nsa)

                pltpu.make_async_copy(s_hbm.at[idxs.at[kk]], rb, sg).wait()
                pltpu.make_async_copy(ad_hbm.at[idxd.at[kk]], ab, sa).wait()

                @plsc.parallel_loop(0, CHUNK, unroll=8)
                def _edge(i):
                    compute_rows(rb, ab, ov, i)

                pltpu.sync_copy(ov, accum.at[idxd.at[kk]], add=True)
            return carry

        lax.fori_loop(0, nq, _ring, 0)
        plsc.subcore_barrier()
        for piece in range(5):
            rows_n = 128 if piece < 4 else 120
            pltpu.sync_copy(
                accum.at[pl.ds(sid * SLAB + piece * 128, rows_n)],
                out_hbm.at[pl.ds(cid * NPAD + sid * SLAB + piece * 128, rows_n)])

    return body


def _compute_rows_l1(rows, adv, ov, i):
    aa = rows[i, pl.ds(0, 16)] + adv[i, pl.ds(0, 16)]
    p = jnp.exp(jnp.maximum(aa, 0.2 * aa))
    ov[i, pl.ds(0, 16)] = p
    for j in range(4):
        ov[i, pl.ds(16 + 16 * j, 16)] = rows[i, pl.ds(16 + 16 * j, 16)] * p


def _compute_rows_l2(rows, adv, ov, i):
    aa = rows[i, pl.ds(0, 16)] + adv[i, pl.ds(0, 16)]
    p = jnp.exp(jnp.maximum(aa, 0.2 * aa))
    ov[i, pl.ds(0, 16)] = p
    ov[i, pl.ds(16, 16)] = rows[i, pl.ds(16, 16)] * p


def _make_sc(width, compute_rows):
    return pl.kernel(
        _make_sc_body(width, compute_rows),
        out_type=jax.ShapeDtypeStruct((NC * NPAD, width), jnp.float32),
        mesh=plsc.VectorSubcoreMesh(core_axis_name="c", subcore_axis_name="s",
                                    num_cores=NC, num_subcores=NS),
        scratch_types=[
            pltpu.VMEM((K0, CHUNK), jnp.int32),
            pltpu.VMEM((K0, CHUNK), jnp.int32),
            pltpu.VMEM((CHUNK, width), jnp.float32),
            pltpu.VMEM((CHUNK, 16), jnp.float32),
            pltpu.VMEM((CHUNK, width), jnp.float32),
            pltpu.VMEM((CHUNK, 16), jnp.float32),
            pltpu.VMEM((CHUNK, width), jnp.float32),
            pltpu.VMEM((CHUNK, 16), jnp.float32),
            pltpu.VMEM((CHUNK, width), jnp.float32),
            pltpu.VMEM((CHUNK, 16), jnp.float32),
            pltpu.VMEM((CHUNK, width), jnp.float32),
            pltpu.VMEM_SHARED((NPAD, width), jnp.float32),
            pltpu.SemaphoreType.DMA,
            pltpu.SemaphoreType.DMA,
            pltpu.SemaphoreType.DMA,
            pltpu.SemaphoreType.DMA,
            pltpu.SemaphoreType.DMA,
            pltpu.SemaphoreType.DMA,
            pltpu.SemaphoreType.DMA,
            pltpu.SemaphoreType.DMA,
        ],
        compiler_params=pltpu.CompilerParams(use_tc_tiling_on_sc=False),
    )


_GRID = NPAD // SLAB


def kernel(x, edge_index, W1, att_src1, att_dst1, b1, W2, att_src2, att_dst2, b2):
    f32 = jnp.float32
    lanes = jnp.arange(F1)
    A_src = jnp.zeros((F1, HEADS), f32).at[lanes, lanes // DH].set(att_src1.reshape(-1))
    A_dst = jnp.zeros((F1, HEADS), f32).at[lanes, lanes // DH].set(att_dst1.reshape(-1))
    t = jnp.arange(F1)
    perm = (t % DH) * HEADS + t // DH
    P1m = jnp.zeros((F1, F1), f32).at[perm, t].set(1.0)
    M1 = jnp.concatenate([A_src, A_src, P1m], axis=1)
    G1 = W1 @ M1
    GAD1 = W1 @ jnp.concatenate([A_dst, A_dst], axis=1)
    ones16 = jnp.ones((1, 16), f32)
    M2 = jnp.concatenate([att_src2.reshape(DOUT, 1) @ ones16,
                          jnp.eye(DOUT, dtype=f32)], axis=1)
    G2 = W2[perm, :] @ M2
    GAD2 = W2[perm, :] @ (att_dst2.reshape(DOUT, 1) @ ones16)
    R = jnp.zeros((HEADS, F1), f32).at[t % DH, t].set(1.0)
    b1r = b1[perm].reshape(1, F1)
    b2r = b2.reshape(1, DOUT)

    loop = jnp.arange(N, dtype=jnp.int32)
    fill = jnp.full((EPAD - N - edge_index.shape[1],), N, jnp.int32)
    src = jnp.concatenate([edge_index[0], loop, fill]).reshape(EPAD // CHUNK, CHUNK)
    dst = jnp.concatenate([edge_index[1], loop, fill]).reshape(EPAD // CHUNK, CHUNK)
    xp = jnp.pad(x, ((0, NPAD - N), (0, 0)))

    S1, AD1 = pl.pallas_call(
        _tc1_body,
        grid=(_GRID,),
        in_specs=[pl.BlockSpec((SLAB, DIN), lambda i: (i, 0)),
                  pl.BlockSpec((DIN, W1R), lambda i: (0, 0)),
                  pl.BlockSpec((DIN, 16), lambda i: (0, 0))],
        out_specs=[pl.BlockSpec((SLAB, W1R), lambda i: (i, 0)),
                   pl.BlockSpec((SLAB, 16), lambda i: (i, 0))],
        out_shape=[jax.ShapeDtypeStruct((NPAD, W1R), f32),
                   jax.ShapeDtypeStruct((NPAD, 16), f32)],
    )(xp, G1, GAD1)

    P1 = _make_sc(W1R, _compute_rows_l1)(src, dst, S1, AD1)

    S2, AD2 = pl.pallas_call(
        _tc2_body,
        grid=(_GRID,),
        in_specs=[pl.BlockSpec((SLAB, W1R), lambda i: (i, 0)),
                  pl.BlockSpec((SLAB, W1R), lambda i: (i + _GRID, 0)),
                  pl.BlockSpec((HEADS, F1), lambda i: (0, 0)),
                  pl.BlockSpec((1, F1), lambda i: (0, 0)),
                  pl.BlockSpec((F1, W2R), lambda i: (0, 0)),
                  pl.BlockSpec((F1, 16), lambda i: (0, 0))],
        out_specs=[pl.BlockSpec((SLAB, W2R), lambda i: (i, 0)),
                   pl.BlockSpec((SLAB, 16), lambda i: (i, 0))],
        out_shape=[jax.ShapeDtypeStruct((NPAD, W2R), f32),
                   jax.ShapeDtypeStruct((NPAD, 16), f32)],
    )(P1, P1, R, b1r, G2, GAD2)

    P2 = _make_sc(W2R, _compute_rows_l2)(src, dst, S2, AD2)

    out = pl.pallas_call(
        _tc3_body,
        grid=(_GRID,),
        in_specs=[pl.BlockSpec((SLAB, W2R), lambda i: (i, 0)),
                  pl.BlockSpec((SLAB, W2R), lambda i: (i + _GRID, 0)),
                  pl.BlockSpec((1, DOUT), lambda i: (0, 0))],
        out_specs=pl.BlockSpec((SLAB, DOUT), lambda i: (i, 0)),
        out_shape=jax.ShapeDtypeStruct((NPAD, DOUT), f32),
    )(P2, P2, b2r)
    return out[:N]


# speedup vs baseline: 1.4257x; 1.0727x over previous
import jax
import jax.numpy as jnp
from jax import lax
from jax.experimental import pallas as pl
from jax.experimental.pallas import tpu as pltpu
from jax.experimental.pallas import tpu_sc as plsc

N = 10000
DIN = 128
HEADS = 8
DH = 8
F1 = HEADS * DH
DOUT = 16

NC, NS = 2, 16
NW = NC * NS
SLAB = 632
NPAD = NS * SLAB
CHUNK = 64
NBUF = 4
PAIR = 328
K0MAX = 256
TOTCH = NS * PAIR
EPAD = (NS * PAIR + K0MAX) * CHUNK

W1R = 80
W2R = 32


def _tc1_body(x_ref, g1_ref, gad_ref, s_ref, ad_ref):
    xb = x_ref[...]
    s_ref[...] = jnp.dot(xb, g1_ref[...], preferred_element_type=jnp.float32)
    ad_ref[...] = jnp.dot(xb, gad_ref[...], preferred_element_type=jnp.float32)


def _tc2_body(pa_ref, pb_ref, r_ref, b1_ref, g2_ref, gad2_ref, s2_ref, ad2_ref):
    A = pa_ref[...] + pb_ref[...]
    dexp = jnp.dot(A[:, 0:8], r_ref[...], preferred_element_type=jnp.float32)
    out1 = jnp.maximum(A[:, 16:80] / (dexp + 1e-16) + b1_ref[...], 0.0)
    s2_ref[...] = jnp.dot(out1, g2_ref[...], preferred_element_type=jnp.float32)
    ad2_ref[...] = jnp.dot(out1, gad2_ref[...], preferred_element_type=jnp.float32)


def _tc3_body(pa_ref, pb_ref, b2_ref, o_ref):
    A = pa_ref[...] + pb_ref[...]
    o2 = A[:, 16:32] / (A[:, 0:1] + 1e-16) + b2_ref[...]
    z = o2 - jnp.max(o2, axis=1, keepdims=True)
    o_ref[...] = z - jnp.log(jnp.sum(jnp.exp(z), axis=1, keepdims=True))


def _make_sc_body(width, compute_rows, K0, K1):
    nslice = width // 16

    def body(src_hbm, dst_hbm, s_hbm, ad_hbm, out_hbm,
             idxs, idxd, rows0, ad0, rows1, ad1, rows2, ad2, rows3, ad3,
             ov, accum, sg0, sa0, sg1, sa1, sg2, sa2, sg3, sa3):
        cid = lax.axis_index("c")
        sid = lax.axis_index("s")
        kbase = sid * PAIR + cid * K0
        cnt = jnp.where(cid == 0, K0, K1)
        nq = jnp.where(cid == 0, K0 // NBUF, K1 // NBUF)
        z16 = jnp.zeros((16,), jnp.float32)

        pltpu.sync_copy(src_hbm.at[pl.ds(kbase, K0)], idxs)
        pltpu.sync_copy(dst_hbm.at[pl.ds(kbase, K0)], idxd)
        pltpu.async_copy(s_hbm.at[idxs.at[0]], rows0, sg0)
        pltpu.async_copy(ad_hbm.at[idxd.at[0]], ad0, sa0)

        @plsc.parallel_loop(0, CHUNK, unroll=8)
        def _zrow(r):
            for j in range(nslice):
                ov[r, pl.ds(j * 16, 16)] = z16
        for piece in range(10):
            rows_n = 64 if piece < 9 else 56
            pltpu.sync_copy(ov.at[pl.ds(0, rows_n)],
                            accum.at[pl.ds(sid * SLAB + piece * 64, rows_n)])
        plsc.subcore_barrier()

        bufs = ((rows0, ad0, sg0, sa0), (rows1, ad1, sg1, sa1),
                (rows2, ad2, sg2, sa2), (rows3, ad3, sg3, sa3))
        for b in range(1, NBUF - 1):
            rb, ab, sg, sa = bufs[b]
            pltpu.async_copy(s_hbm.at[idxs.at[b]], rb, sg)
            pltpu.async_copy(ad_hbm.at[idxd.at[b]], ab, sa)

        def _ring(kq, carry):
            k0 = kq * NBUF
            for b in range(NBUF):
                kk = k0 + b
                rb, ab, sg, sa = bufs[b]
                nrb, nab, nsg, nsa = bufs[(b + NBUF - 1) % NBUF]

                @pl.when(kk + NBUF - 1 < cnt)
                def _issue():
                    pltpu.async_copy(s_hbm.at[idxs.at[kk + NBUF - 1]], nrb, nsg)
                    pltpu.async_copy(ad_hbm.at[idxd.at[kk + NBUF - 1]], nab, nsa)

                pltpu.make_async_copy(s_hbm.at[idxs.at[kk]], rb, sg).wait()
                pltpu.make_async_copy(ad_hbm.at[idxd.at[kk]], ab, sa).wait()

                @plsc.parallel_loop(0, CHUNK, unroll=8)
                def _edge(i):
                    compute_rows(rb, ab, ov, i)

                pltpu.sync_copy(ov, accum.at[idxd.at[kk]], add=True)
            return carry

        lax.fori_loop(0, nq, _ring, 0)
        plsc.subcore_barrier()
        for piece in range(5):
            rows_n = 128 if piece < 4 else 120
            pltpu.sync_copy(
                accum.at[pl.ds(sid * SLAB + piece * 128, rows_n)],
                out_hbm.at[pl.ds(cid * NPAD + sid * SLAB + piece * 128, rows_n)])

    return body


def _compute_rows_l1(rows, adv, ov, i):
    aa = rows[i, pl.ds(0, 16)] + adv[i, pl.ds(0, 16)]
    p = jnp.exp(jnp.maximum(aa, 0.2 * aa))
    ov[i, pl.ds(0, 16)] = p
    for j in range(4):
        ov[i, pl.ds(16 + 16 * j, 16)] = rows[i, pl.ds(16 + 16 * j, 16)] * p


def _compute_rows_l2(rows, adv, ov, i):
    aa = rows[i, pl.ds(0, 16)] + adv[i, pl.ds(0, 16)]
    p = jnp.exp(jnp.maximum(aa, 0.2 * aa))
    ov[i, pl.ds(0, 16)] = p
    ov[i, pl.ds(16, 16)] = rows[i, pl.ds(16, 16)] * p


def _make_sc(width, compute_rows, k0, k1):
    return pl.kernel(
        _make_sc_body(width, compute_rows, k0, k1),
        out_type=jax.ShapeDtypeStruct((NC * NPAD, width), jnp.float32),
        mesh=plsc.VectorSubcoreMesh(core_axis_name="c", subcore_axis_name="s",
                                    num_cores=NC, num_subcores=NS),
        scratch_types=[
            pltpu.VMEM((k0, CHUNK), jnp.int32),
            pltpu.VMEM((k0, CHUNK), jnp.int32),
            pltpu.VMEM((CHUNK, width), jnp.float32),
            pltpu.VMEM((CHUNK, 16), jnp.float32),
            pltpu.VMEM((CHUNK, width), jnp.float32),
            pltpu.VMEM((CHUNK, 16), jnp.float32),
            pltpu.VMEM((CHUNK, width), jnp.float32),
            pltpu.VMEM((CHUNK, 16), jnp.float32),
            pltpu.VMEM((CHUNK, width), jnp.float32),
            pltpu.VMEM((CHUNK, 16), jnp.float32),
            pltpu.VMEM((CHUNK, width), jnp.float32),
            pltpu.VMEM_SHARED((NPAD, width), jnp.float32),
            pltpu.SemaphoreType.DMA,
            pltpu.SemaphoreType.DMA,
            pltpu.SemaphoreType.DMA,
            pltpu.SemaphoreType.DMA,
            pltpu.SemaphoreType.DMA,
            pltpu.SemaphoreType.DMA,
            pltpu.SemaphoreType.DMA,
            pltpu.SemaphoreType.DMA,
        ],
        compiler_params=pltpu.CompilerParams(use_tc_tiling_on_sc=False),
    )


_GRID = NPAD // SLAB


def kernel(x, edge_index, W1, att_src1, att_dst1, b1, W2, att_src2, att_dst2, b2):
    f32 = jnp.float32
    lanes = jnp.arange(F1)
    A_src = jnp.zeros((F1, HEADS), f32).at[lanes, lanes // DH].set(att_src1.reshape(-1))
    A_dst = jnp.zeros((F1, HEADS), f32).at[lanes, lanes // DH].set(att_dst1.reshape(-1))
    t = jnp.arange(F1)
    perm = (t % DH) * HEADS + t // DH
    P1m = jnp.zeros((F1, F1), f32).at[perm, t].set(1.0)
    M1 = jnp.concatenate([A_src, A_src, P1m], axis=1)
    G1 = W1 @ M1
    GAD1 = W1 @ jnp.concatenate([A_dst, A_dst], axis=1)
    ones16 = jnp.ones((1, 16), f32)
    M2 = jnp.concatenate([att_src2.reshape(DOUT, 1) @ ones16,
                          jnp.eye(DOUT, dtype=f32)], axis=1)
    G2 = W2[perm, :] @ M2
    GAD2 = W2[perm, :] @ (att_dst2.reshape(DOUT, 1) @ ones16)
    R = jnp.zeros((HEADS, F1), f32).at[t % DH, t].set(1.0)
    b1r = b1[perm].reshape(1, F1)
    b2r = b2.reshape(1, DOUT)

    loop = jnp.arange(N, dtype=jnp.int32)
    fill = jnp.full((EPAD - N - edge_index.shape[1],), N, jnp.int32)
    src = jnp.concatenate([edge_index[0], loop, fill]).reshape(EPAD // CHUNK, CHUNK)
    dst = jnp.concatenate([edge_index[1], loop, fill]).reshape(EPAD // CHUNK, CHUNK)
    xp = jnp.pad(x, ((0, NPAD - N), (0, 0)))

    S1, AD1 = pl.pallas_call(
        _tc1_body,
        grid=(_GRID,),
        in_specs=[pl.BlockSpec((SLAB, DIN), lambda i: (i, 0)),
                  pl.BlockSpec((DIN, W1R), lambda i: (0, 0)),
                  pl.BlockSpec((DIN, 16), lambda i: (0, 0))],
        out_specs=[pl.BlockSpec((SLAB, W1R), lambda i: (i, 0)),
                   pl.BlockSpec((SLAB, 16), lambda i: (i, 0))],
        out_shape=[jax.ShapeDtypeStruct((NPAD, W1R), f32),
                   jax.ShapeDtypeStruct((NPAD, 16), f32)],
    )(xp, G1, GAD1)

    P1 = _make_sc(W1R, _compute_rows_l1, 256, 72)(src, dst, S1, AD1)

    S2, AD2 = pl.pallas_call(
        _tc2_body,
        grid=(_GRID,),
        in_specs=[pl.BlockSpec((SLAB, W1R), lambda i: (i, 0)),
                  pl.BlockSpec((SLAB, W1R), lambda i: (i + _GRID, 0)),
                  pl.BlockSpec((HEADS, F1), lambda i: (0, 0)),
                  pl.BlockSpec((1, F1), lambda i: (0, 0)),
                  pl.BlockSpec((F1, W2R), lambda i: (0, 0)),
                  pl.BlockSpec((F1, 16), lambda i: (0, 0))],
        out_specs=[pl.BlockSpec((SLAB, W2R), lambda i: (i, 0)),
                   pl.BlockSpec((SLAB, 16), lambda i: (i, 0))],
        out_shape=[jax.ShapeDtypeStruct((NPAD, W2R), f32),
                   jax.ShapeDtypeStruct((NPAD, 16), f32)],
    )(P1, P1, R, b1r, G2, GAD2)

    P2 = _make_sc(W2R, _compute_rows_l2, 232, 96)(src, dst, S2, AD2)

    out = pl.pallas_call(
        _tc3_body,
        grid=(_GRID,),
        in_specs=[pl.BlockSpec((SLAB, W2R), lambda i: (i, 0)),
                  pl.BlockSpec((SLAB, W2R), lambda i: (i + _GRID, 0)),
                  pl.BlockSpec((1, DOUT), lambda i: (0, 0))],
        out_specs=pl.BlockSpec((SLAB, DOUT), lambda i: (i, 0)),
        out_shape=jax.ShapeDtypeStruct((NPAD, DOUT), f32),
    )(P2, P2, b2r)
    return out[:N]
